# baseline (speedup 1.0000x reference)
import jax
import jax.numpy as jnp
from jax import lax
from jax.experimental import pallas as pl
from jax.experimental.pallas import tpu as pltpu
from jax.experimental.pallas import tpu_sc as plsc

N = 10000
E = 320000
D = 128
DH = 64
HV = DH // 16

NC = 2
NS = 16
L = 16
NW = NC * NS
EPT = E // NW
BLK = 128
NB = -(-EPT // BLK)
EPTP = NB * BLK
NP = 10240
RPS = NP // NS
GE = 16
GPB = BLK // GE

_F32 = jnp.float32


EPC = E // NS
NPW = NP // NW


def _rsqrt16(v):
    y = plsc.bitcast(
        jnp.int32(0x5F3759DF) - (plsc.bitcast(v, jnp.int32) >> 1), _F32
    )
    for _ in range(3):
        y = y * (1.5 - 0.5 * v * y * y)
    return y


def _deg_body(ei_hbm, ew_hbm, out_hbm, dst_v, ew_v, acc_v, red_v, bc_v, buf_sh):
    c = lax.axis_index("c")
    s = lax.axis_index("s")
    w = c * NS + s
    pltpu.sync_copy(ei_hbm.at[pl.ds(E + s * EPC, EPC)], dst_v)
    pltpu.sync_copy(ew_hbm.at[pl.ds(s * EPC, EPC)], ew_v)

    def _zero(i, carry):
        acc_v[pl.ds(i * L, L)] = jnp.zeros((L,), _F32)
        return carry

    lax.fori_loop(0, NP // L, _zero, 0)

    def _acc(i, carry):
        idx = dst_v[pl.ds(i * L, L)]
        val = ew_v[pl.ds(i * L, L)]
        plsc.addupdate_scatter(acc_v, [idx], val)
        return carry

    lax.fori_loop(0, EPC // L, _acc, 0)

    pltpu.sync_copy(acc_v, buf_sh.at[s])
    plsc.subcore_barrier()

    pltpu.sync_copy(buf_sh.at[:, pl.ds(w * NPW, NPW)], red_v)

    def _red(j, carry):
        deg = red_v[0, pl.ds(j * L, L)]
        for r in range(1, NS):
            deg = deg + red_v[r, pl.ds(j * L, L)]
        dis = _rsqrt16(deg + 1.0)
        for e in range(L):
            dv = dis[e]
            for f in range(D // L):
                bc_v[j * L + e, pl.ds(f * L, L)] = jnp.full((L,), dv, _F32)
        return carry

    lax.fori_loop(0, NPW // L, _red, 0)
    pltpu.sync_copy(bc_v, out_hbm.at[pl.ds(w * NPW, NPW)])


NSL = 3


def _prop_body(u_hbm, ei_hbm, ew_hbm, out_hbm,
               src2_v, dst_v, ew_v, g_bufs, s_bufs,
               gsem0, gsem1, gsem2, ssem0, ssem1, ssem2,
               acc_sh):
    c = lax.axis_index("c")
    s = lax.axis_index("s")
    w = s * NC + c
    pltpu.sync_copy(ei_hbm.at[pl.ds(w * EPT, EPT)], src2_v.at[pl.ds(0, EPT)])
    pltpu.sync_copy(ei_hbm.at[pl.ds(E + w * EPT, EPT)], dst_v.at[pl.ds(0, EPT)])
    pltpu.sync_copy(ew_hbm.at[pl.ds(w * EPT, EPT)], ew_v.at[pl.ds(0, EPT)])

    for i in range((EPTP - EPT) // L):
        dst_v[pl.ds(EPT + i * L, L)] = jnp.zeros((L,), jnp.int32)
        ew_v[pl.ds(EPT + i * L, L)] = jnp.zeros((L,), _F32)
        src2_v[pl.ds(EPT + i * L, L)] = jnp.zeros((L,), jnp.int32)

    gsems = (gsem0, gsem1, gsem2)
    ssems = (ssem0, ssem1, ssem2)

    def _zrow(i, carry):
        s_bufs[0, i // HV, pl.ds((i % HV) * L, L)] = jnp.zeros((L,), _F32)
        return carry

    for half in range(2):
        def _ix(i, carry):
            sv = src2_v[pl.ds(i * L, L)]
            src2_v[pl.ds(i * L, L)] = sv * 2 + half if half == 0 else sv + 1
            return carry

        lax.fori_loop(0, EPT // L, _ix, 0)

        lax.fori_loop(0, BLK * HV, _zrow, 0)
        for k in range(RPS // BLK):
            pltpu.sync_copy(
                s_bufs.at[0], acc_sh.at[pl.ds(s * RPS + k * BLK, BLK)]
            )
        plsc.subcore_barrier()

        def _start_g(b, slot):
            pltpu.async_copy(
                u_hbm.at[src2_v.at[pl.ds(b * BLK, BLK)]], g_bufs.at[slot],
                gsems[slot],
            )

        def _wait_g(slot):
            pltpu.make_async_copy(
                u_hbm.at[src2_v.at[pl.ds(0, BLK)]], g_bufs.at[slot],
                gsems[slot],
            ).wait()

        def _start_s(b, slot):
            pltpu.async_copy(
                s_bufs.at[slot], acc_sh.at[dst_v.at[pl.ds(b * BLK, BLK)]],
                ssems[slot], add=True,
            )

        def _wait_s(slot):
            pltpu.make_async_copy(
                s_bufs.at[slot], acc_sh.at[dst_v.at[pl.ds(0, BLK)]],
                ssems[slot]
            ).wait()

        def _scale(b, slot):
            def _grp(g, carry):
                ewv = ew_v[pl.ds((b * GPB + g) * GE, GE)]
                for e in range(GE):
                    wv = ewv[e]
                    row = g * GE + e
                    for f in range(HV):
                        s_bufs[slot, row, pl.ds(f * L, L)] = (
                            g_bufs[slot, row, pl.ds(f * L, L)] * wv
                        )
                return carry

            lax.fori_loop(0, GPB, _grp, 0)

        for k in range(NSL):
            _start_g(k, k)
        for k in range(NSL):
            _wait_g(k)
            _scale(k, k)
            _start_s(k, k)
            _start_g(k + NSL, k)

        def _quad(j, carry):
            b0 = j * NSL
            for k in range(NSL):
                b = b0 + k
                _wait_g(k)
                _wait_s(k)
                _scale(b, k)
                _start_s(b, k)

                @pl.when(b + NSL < NB)
                def _():
                    _start_g(b + NSL, k)
            return carry

        lax.fori_loop(1, NB // NSL, _quad, 0)
        for k in range(NB - NSL * (NB // NSL)):
            _wait_g(k)
            _wait_s(k)
            _scale(NSL * (NB // NSL) + k, k)
            _start_s(NSL * (NB // NSL) + k, k)
        for k in range(NSL):
            _wait_s(k)

        plsc.subcore_barrier()
        pltpu.sync_copy(
            acc_sh.at[pl.ds(s * RPS, RPS)],
            out_hbm.at[c, half, pl.ds(s * RPS, RPS)],
        )


def _make_sc_calls():
    mesh = plsc.VectorSubcoreMesh(
        core_axis_name="c", subcore_axis_name="s", num_cores=NC, num_subcores=NS
    )
    deg_call = pl.kernel(
        _deg_body,
        out_type=jax.ShapeDtypeStruct((NP, D), _F32),
        mesh=mesh,
        compiler_params=pltpu.CompilerParams(
            needs_layout_passes=False, use_tc_tiling_on_sc=False
        ),
        scratch_types=[
            pltpu.VMEM((EPC,), jnp.int32),
            pltpu.VMEM((EPC,), _F32),
            pltpu.VMEM((NP,), _F32),
            pltpu.VMEM((NS, NPW), _F32),
            pltpu.VMEM((NPW, D), _F32),
            pltpu.VMEM_SHARED((NS, NP), _F32),
        ],
    )
    prop_call = pl.kernel(
        _prop_body,
        out_type=jax.ShapeDtypeStruct((NC, 2, NP, DH), _F32),
        mesh=mesh,
        compiler_params=pltpu.CompilerParams(use_tc_tiling_on_sc=False),
        scratch_types=[
            pltpu.VMEM((EPTP,), jnp.int32),
            pltpu.VMEM((EPTP,), jnp.int32),
            pltpu.VMEM((EPTP,), _F32),
            pltpu.VMEM((NSL, BLK, DH), _F32),
            pltpu.VMEM((NSL, BLK, DH), _F32),
            pltpu.SemaphoreType.DMA,
            pltpu.SemaphoreType.DMA,
            pltpu.SemaphoreType.DMA,
            pltpu.SemaphoreType.DMA,
            pltpu.SemaphoreType.DMA,
            pltpu.SemaphoreType.DMA,
            pltpu.VMEM_SHARED((NP, DH), _F32),
        ],
    )
    return deg_call, prop_call


_RB = 1000
_NRB = N // _RB


def _m1_body(x_ref, w_ref, dis_ref, u_ref):
    h = jnp.dot(x_ref[...], w_ref[...], preferred_element_type=_F32)
    u_ref[...] = h * dis_ref[...]


def _post_body(p_ref, u_ref, dis_ref, b_ref, g_ref, be_ref, w_ref, u2_ref,
               z_sc, st_sc, *, with_matmul):
    ph = pl.program_id(0)
    i = pl.program_id(1)

    @pl.when(jnp.logical_and(ph == 0, i == 0))
    def _():
        st_sc[...] = jnp.zeros_like(st_sc)

    @pl.when(ph == 0)
    def _():
        su = jnp.concatenate(
            [p_ref[0, 0] + p_ref[1, 0],
             p_ref[0, 1] + p_ref[1, 1]], axis=1) + u_ref[...]
        z = su * dis_ref[...] + b_ref[...]
        z_sc[pl.ds(i * _RB, _RB), :] = z
        s1 = jnp.sum(z, axis=0, keepdims=True)
        s2 = jnp.sum(z * z, axis=0, keepdims=True)
        st_sc[...] += jnp.concatenate([s1, s2], axis=0)

    @pl.when(ph == 1)
    def _():
        st = st_sc[...]
        mu = st[0:1] * (1.0 / N)
        var = st[1:2] * (1.0 / N) - mu * mu
        z = z_sc[pl.ds(i * _RB, _RB), :]
        a = g_ref[...] * (z - mu) * lax.rsqrt(var + 1e-5) + be_ref[...]
        a = jnp.maximum(a, 0.0)
        if with_matmul:
            h2 = jnp.dot(a, w_ref[...], preferred_element_type=_F32)
            u2_ref[...] = h2 * dis_ref[...]
        else:
            u2_ref[...] = a


def kernel(x, edge_index, edge_weight, W1, b1, bn1_g, bn1_b, W2, b2, bn2_g, bn2_b):
    ei_flat = edge_index.reshape(2 * E)

    deg_call, prop_call = _make_sc_calls()

    dis2d = deg_call(ei_flat, edge_weight)

    u1 = pl.pallas_call(
        _m1_body,
        grid=(_NRB,),
        in_specs=[
            pl.BlockSpec((_RB, D), lambda i: (i, 0)),
            pl.BlockSpec((D, D), lambda i: (0, 0)),
            pl.BlockSpec((_RB, D), lambda i: (i, 0)),
        ],
        out_specs=pl.BlockSpec((_RB, D), lambda i: (i, 0)),
        out_shape=jax.ShapeDtypeStruct((N, D), _F32),
    )(x, W1, dis2d)

    def post(part, u, bias, g, be, w, with_matmul):
        import functools
        body = functools.partial(_post_body, with_matmul=with_matmul)
        out_specs = pl.BlockSpec((_RB, D), lambda p, i: (i * p, 0))
        out_shape = jax.ShapeDtypeStruct((N, D), _F32)
        return pl.pallas_call(
            body,
            grid=(2, _NRB),
            in_specs=[
                pl.BlockSpec((NC, 2, _RB, DH),
                             lambda p, i: (0, 0, i * (1 - p), 0)),
                pl.BlockSpec((_RB, D), lambda p, i: (i * (1 - p), 0)),
                pl.BlockSpec((_RB, D), lambda p, i: (i, 0)),
                pl.BlockSpec((1, D), lambda p, i: (0, 0)),
                pl.BlockSpec((1, D), lambda p, i: (0, 0)),
                pl.BlockSpec((1, D), lambda p, i: (0, 0)),
                pl.BlockSpec((D, D), lambda p, i: (0, 0)),
            ],
            out_specs=out_specs,
            out_shape=out_shape,
            scratch_shapes=[
                pltpu.VMEM((N, D), _F32),
                pltpu.VMEM((2, D), _F32),
            ],
        )(part, u, dis2d, bias.reshape(1, D), g.reshape(1, D),
          be.reshape(1, D), w)

    part1 = prop_call(u1.reshape(2 * N, DH), ei_flat, edge_weight)
    u2 = post(part1, u1, b1, bn1_g, bn1_b, W2, True)

    part2 = prop_call(u2.reshape(2 * N, DH), ei_flat, edge_weight)
    out = post(part2, u2, b2, bn2_g, bn2_b, W2, False)
    return out


# speedup vs baseline: 1.6962x; 1.6962x over previous
import jax
import jax.numpy as jnp
from jax import lax
from jax.experimental import pallas as pl
from jax.experimental.pallas import tpu as pltpu
from jax.experimental.pallas import tpu_sc as plsc

N = 10000
E = 320000
D = 128
DH = 64
HV = DH // 16

NC = 2
NS = 16
L = 16
NW = NC * NS
EPT = E // NW
BLK = 80
NB = EPT // BLK
NP = 10240
RPS = NP // NS
GE = 16
GPB = BLK // GE

_F32 = jnp.float32


EPC = E // NS
NPW = NP // NW


def _rsqrt16(v):
    y = plsc.bitcast(
        jnp.int32(0x5F3759DF) - (plsc.bitcast(v, jnp.int32) >> 1), _F32
    )
    for _ in range(3):
        y = y * (1.5 - 0.5 * v * y * y)
    return y


def _deg_body(ei_hbm, ew_hbm, out_hbm, dst_v, ew_v, acc_v, red_v, bc_v, buf_sh):
    c = lax.axis_index("c")
    s = lax.axis_index("s")
    w = c * NS + s
    pltpu.sync_copy(ei_hbm.at[pl.ds(E + s * EPC, EPC)], dst_v)
    pltpu.sync_copy(ew_hbm.at[pl.ds(s * EPC, EPC)], ew_v)

    def _zero(i, carry):
        acc_v[pl.ds(i * L, L)] = jnp.zeros((L,), _F32)
        return carry

    lax.fori_loop(0, NP // L, _zero, 0)

    def _acc(i, carry):
        idx = dst_v[pl.ds(i * L, L)]
        val = ew_v[pl.ds(i * L, L)]
        plsc.addupdate_scatter(acc_v, [idx], val)
        return carry

    lax.fori_loop(0, EPC // L, _acc, 0)

    pltpu.sync_copy(acc_v, buf_sh.at[s])
    plsc.subcore_barrier()

    pltpu.sync_copy(buf_sh.at[:, pl.ds(w * NPW, NPW)], red_v)

    def _red(j, carry):
        deg = red_v[0, pl.ds(j * L, L)]
        for r in range(1, NS):
            deg = deg + red_v[r, pl.ds(j * L, L)]
        dis = _rsqrt16(deg + 1.0)
        for e in range(L):
            dv = dis[e]
            for f in range(D // L):
                bc_v[j * L + e, pl.ds(f * L, L)] = jnp.full((L,), dv, _F32)
        return carry

    lax.fori_loop(0, NPW // L, _red, 0)
    pltpu.sync_copy(bc_v, out_hbm.at[pl.ds(w * NPW, NPW)])


NSL = 4


def _prop_body(u_hbm, ei_hbm, ew_hbm, out_hbm,
               src_v, src2_v, dst_v, ew_v, g_bufs, s_bufs,
               gsem0, gsem1, gsem2, gsem3, ssem0, ssem1, ssem2, ssem3,
               acc_sh):
    c = lax.axis_index("c")
    s = lax.axis_index("s")
    w = s * NC + c
    pltpu.sync_copy(ei_hbm.at[pl.ds(w * EPT, EPT)], src_v)
    pltpu.sync_copy(ei_hbm.at[pl.ds(E + w * EPT, EPT)], dst_v)
    pltpu.sync_copy(ew_hbm.at[pl.ds(w * EPT, EPT)], ew_v)

    gsems = (gsem0, gsem1, gsem2, gsem3)
    ssems = (ssem0, ssem1, ssem2, ssem3)

    def _zrow(i, carry):
        s_bufs[0, i // HV, pl.ds((i % HV) * L, L)] = jnp.zeros((L,), _F32)
        return carry

    for half in range(2):
        def _ix(i, carry):
            sv = src_v[pl.ds(i * L, L)]
            src2_v[pl.ds(i * L, L)] = sv * 2 + half
            return carry

        lax.fori_loop(0, EPT // L, _ix, 0)

        lax.fori_loop(0, BLK * HV, _zrow, 0)
        for k in range(RPS // BLK):
            pltpu.sync_copy(
                s_bufs.at[0], acc_sh.at[pl.ds(s * RPS + k * BLK, BLK)]
            )
        plsc.subcore_barrier()

        def _start_g(b, slot):
            pltpu.async_copy(
                u_hbm.at[src2_v.at[pl.ds(b * BLK, BLK)]], g_bufs.at[slot],
                gsems[slot],
            )

        def _wait_g(slot):
            pltpu.make_async_copy(
                u_hbm.at[src2_v.at[pl.ds(0, BLK)]], g_bufs.at[slot],
                gsems[slot],
            ).wait()

        def _start_s(b, slot):
            pltpu.async_copy(
                s_bufs.at[slot], acc_sh.at[dst_v.at[pl.ds(b * BLK, BLK)]],
                ssems[slot], add=True,
            )

        def _wait_s(slot):
            pltpu.make_async_copy(
                s_bufs.at[slot], acc_sh.at[dst_v.at[pl.ds(0, BLK)]],
                ssems[slot]
            ).wait()

        def _scale(b, slot):
            def _grp(g, carry):
                ewv = ew_v[pl.ds((b * GPB + g) * GE, GE)]
                for e in range(GE):
                    wv = ewv[e]
                    row = g * GE + e
                    for f in range(HV):
                        s_bufs[slot, row, pl.ds(f * L, L)] = (
                            g_bufs[slot, row, pl.ds(f * L, L)] * wv
                        )
                return carry

            lax.fori_loop(0, GPB, _grp, 0)

        for k in range(NSL):
            _start_g(k, k)
        for k in range(NSL):
            _wait_g(k)
            _scale(k, k)
            _start_s(k, k)
            _start_g(k + NSL, k)

        def _quad(j, carry):
            b0 = j * NSL
            for k in range(NSL):
                b = b0 + k
                _wait_g(k)
                _wait_s(k)
                _scale(b, k)
                _start_s(b, k)

                @pl.when(b + NSL < NB)
                def _():
                    _start_g(b + NSL, k)
            return carry

        lax.fori_loop(1, NB // NSL, _quad, 0)
        _wait_g(0)
        _wait_s(0)
        _scale(NB - 1, 0)
        _start_s(NB - 1, 0)
        for k in range(NSL):
            _wait_s(k)

        plsc.subcore_barrier()
        pltpu.sync_copy(
            acc_sh.at[pl.ds(s * RPS, RPS)],
            out_hbm.at[c, half, pl.ds(s * RPS, RPS)],
        )


def _make_sc_calls():
    mesh = plsc.VectorSubcoreMesh(
        core_axis_name="c", subcore_axis_name="s", num_cores=NC, num_subcores=NS
    )
    deg_call = pl.kernel(
        _deg_body,
        out_type=jax.ShapeDtypeStruct((NP, D), _F32),
        mesh=mesh,
        compiler_params=pltpu.CompilerParams(
            needs_layout_passes=False, use_tc_tiling_on_sc=False
        ),
        scratch_types=[
            pltpu.VMEM((EPC,), jnp.int32),
            pltpu.VMEM((EPC,), _F32),
            pltpu.VMEM((NP,), _F32),
            pltpu.VMEM((NS, NPW), _F32),
            pltpu.VMEM((NPW, D), _F32),
            pltpu.VMEM_SHARED((NS, NP), _F32),
        ],
    )
    prop_call = pl.kernel(
        _prop_body,
        out_type=jax.ShapeDtypeStruct((NC, 2, NP, DH), _F32),
        mesh=mesh,
        compiler_params=pltpu.CompilerParams(use_tc_tiling_on_sc=False),
        scratch_types=[
            pltpu.VMEM((EPT,), jnp.int32),
            pltpu.VMEM((EPT,), jnp.int32),
            pltpu.VMEM((EPT,), jnp.int32),
            pltpu.VMEM((EPT,), _F32),
            pltpu.VMEM((NSL, BLK, DH), _F32),
            pltpu.VMEM((NSL, BLK, DH), _F32),
            pltpu.SemaphoreType.DMA,
            pltpu.SemaphoreType.DMA,
            pltpu.SemaphoreType.DMA,
            pltpu.SemaphoreType.DMA,
            pltpu.SemaphoreType.DMA,
            pltpu.SemaphoreType.DMA,
            pltpu.SemaphoreType.DMA,
            pltpu.SemaphoreType.DMA,
            pltpu.VMEM_SHARED((NP, DH), _F32),
        ],
    )
    return deg_call, prop_call


_RB = 1000
_NRB = N // _RB


def _m1_body(x_ref, w_ref, dis_ref, u_ref):
    h = jnp.dot(x_ref[...], w_ref[...], preferred_element_type=_F32)
    u_ref[...] = h * dis_ref[...]


def _post_body(p_ref, u_ref, dis_ref, b_ref, g_ref, be_ref, w_ref, u2_ref,
               z_sc, st_sc, *, with_matmul):
    ph = pl.program_id(0)
    i = pl.program_id(1)

    @pl.when(jnp.logical_and(ph == 0, i == 0))
    def _():
        st_sc[...] = jnp.zeros_like(st_sc)

    @pl.when(ph == 0)
    def _():
        su = jnp.concatenate(
            [p_ref[0, 0] + p_ref[1, 0],
             p_ref[0, 1] + p_ref[1, 1]], axis=1) + u_ref[...]
        z = su * dis_ref[...] + b_ref[...]
        z_sc[pl.ds(i * _RB, _RB), :] = z
        s1 = jnp.sum(z, axis=0, keepdims=True)
        s2 = jnp.sum(z * z, axis=0, keepdims=True)
        st_sc[...] += jnp.concatenate([s1, s2], axis=0)

    @pl.when(ph == 1)
    def _():
        st = st_sc[...]
        mu = st[0:1] * (1.0 / N)
        var = st[1:2] * (1.0 / N) - mu * mu
        z = z_sc[pl.ds(i * _RB, _RB), :]
        a = g_ref[...] * (z - mu) * lax.rsqrt(var + 1e-5) + be_ref[...]
        a = jnp.maximum(a, 0.0)
        if with_matmul:
            h2 = jnp.dot(a, w_ref[...], preferred_element_type=_F32)
            u2_ref[...] = h2 * dis_ref[...]
        else:
            u2_ref[...] = a


def kernel(x, edge_index, edge_weight, W1, b1, bn1_g, bn1_b, W2, b2, bn2_g, bn2_b):
    ei_flat = edge_index.reshape(2 * E)

    deg_call, prop_call = _make_sc_calls()

    dis2d = deg_call(ei_flat, edge_weight)

    u1 = pl.pallas_call(
        _m1_body,
        grid=(_NRB,),
        in_specs=[
            pl.BlockSpec((_RB, D), lambda i: (i, 0)),
            pl.BlockSpec((D, D), lambda i: (0, 0)),
            pl.BlockSpec((_RB, D), lambda i: (i, 0)),
        ],
        out_specs=pl.BlockSpec((_RB, D), lambda i: (i, 0)),
        out_shape=jax.ShapeDtypeStruct((N, D), _F32),
    )(x, W1, dis2d)

    def post(part, u, bias, g, be, w, with_matmul):
        import functools
        body = functools.partial(_post_body, with_matmul=with_matmul)
        out_specs = pl.BlockSpec((_RB, D), lambda p, i: (i * p, 0))
        out_shape = jax.ShapeDtypeStruct((N, D), _F32)
        return pl.pallas_call(
            body,
            grid=(2, _NRB),
            in_specs=[
                pl.BlockSpec((NC, 2, _RB, DH),
                             lambda p, i: (0, 0, i * (1 - p), 0)),
                pl.BlockSpec((_RB, D), lambda p, i: (i * (1 - p), 0)),
                pl.BlockSpec((_RB, D), lambda p, i: (i, 0)),
                pl.BlockSpec((1, D), lambda p, i: (0, 0)),
                pl.BlockSpec((1, D), lambda p, i: (0, 0)),
                pl.BlockSpec((1, D), lambda p, i: (0, 0)),
                pl.BlockSpec((D, D), lambda p, i: (0, 0)),
            ],
            out_specs=out_specs,
            out_shape=out_shape,
            scratch_shapes=[
                pltpu.VMEM((N, D), _F32),
                pltpu.VMEM((2, D), _F32),
            ],
        )(part, u, dis2d, bias.reshape(1, D), g.reshape(1, D),
          be.reshape(1, D), w)

    part1 = prop_call(u1.reshape(2 * N, DH), ei_flat, edge_weight)
    u2 = post(part1, u1, b1, bn1_g, bn1_b, W2, True)

    part2 = prop_call(u2.reshape(2 * N, DH), ei_flat, edge_weight)
    out = post(part2, u2, b2, bn2_g, bn2_b, W2, False)
    return out
